# baseline (speedup 1.0000x reference)
import jax
import jax.numpy as jnp
from jax import lax
from jax.experimental import pallas as pl
from jax.experimental.pallas import tpu as pltpu
from jax.experimental.pallas import tpu_sc as plsc

_B = 16384
_K = 20
_D_EV = 300
_D_CLS = 64
_D_TIME = 36
_D_U = 400
_NCHUNK = _D_U // 16
_N_EV = 100000
_N_CTX = 100000
_NWORD_CHUNKS = 16
_NB_CHUNKS = 9

_NC = 2
_NS = 16
_NW = _NC * _NS
_BW = _B // _NW
_C = 4
_STEPS = _BW // _C


def _lanes():
    return lax.iota(jnp.int32, 16)


def _load_u_regs(ev_g, cls_r, time_r, r):
    li = _lanes()
    row = jnp.full((16,), r, jnp.int32)
    regs = []
    for c in range(_NCHUNK):
        d0 = c * 16
        if c == 18:
            a = plsc.load_gather(ev_g[2], [row, d0 - 256 + li])
            b = plsc.load_gather(cls_r, [row, jnp.maximum(d0 + li - _D_EV, 0)])
            regs.append(jnp.where(li < 12, a, b))
        elif c == 22:
            a = plsc.load_gather(cls_r, [row, jnp.minimum(d0 - _D_EV + li, _D_CLS - 1)])
            b = plsc.load_gather(time_r, [row, jnp.maximum(d0 + li - (_D_EV + _D_CLS), 0)])
            regs.append(jnp.where(li < 12, a, b))
        elif d0 + 16 <= _D_EV:
            regs.append(plsc.load_gather(ev_g[c // 8], [row, (c % 8) * 16 + li]))
        elif d0 + 16 <= _D_EV + _D_CLS:
            regs.append(plsc.load_gather(cls_r, [row, d0 - _D_EV + li]))
        else:
            regs.append(plsc.load_gather(time_r, [row, d0 - (_D_EV + _D_CLS) + li]))
    return regs


def _dot400w(u_regs, w_g, vrow):
    li = _lanes()
    row = jnp.full((16,), vrow, jnp.int32)
    accs = [jnp.zeros((16,), jnp.float32) for _ in range(4)]
    for w in range(_NWORD_CHUNKS):
        wv = plsc.load_gather(w_g[w // 8], [row, (w % 8) * 16 + li])
        wi = plsc.bitcast(wv, jnp.int32)
        a = plsc.bitcast(wi << 16, jnp.float32)
        accs[w % 4] = accs[w % 4] + u_regs[w] * a
        if w < _NB_CHUNKS:
            accs[(w + 2) % 4] = accs[(w + 2) % 4] + u_regs[16 + w] * wv
    return (accs[0] + accs[1]) + (accs[2] + accs[3])


_QS = 4
_GRP_F32 = _QS * _C * 21 * 16
_W_SPAN = _BW * 21 * 16
_OUT_N = _B * 21 * 16


def _store_acc(sbuf, d, acc):
    li = _lanes()
    plsc.store_scatter(sbuf, [d * 16 + li], acc)


def _sc_body(ev_i_h, cls_i_h, time_i_h, pos_i_h, neg_i_h,
             ev0, ev1, ev2, clsemb, temb, ctxw0, ctxw1, out,
             idx_ev, idx_cls, idx_time, idx_pos, idx_neg,
             *bufs_flat):
    sbufA = bufs_flat[-6]
    sbufB = bufs_flat[-5]
    ssemA = bufs_flat[-4]
    ssemB = bufs_flat[-3]
    sem0 = bufs_flat[-2]
    sem1 = bufs_flat[-1]
    nper = (len(bufs_flat) - 6) // 2
    sets = (tuple(bufs_flat[:nper]) + (sem0,),
            tuple(bufs_flat[nper:2 * nper]) + (sem1,))

    cid = lax.axis_index("c")
    sid = lax.axis_index("s")
    wid = sid * _NC + cid

    pltpu.sync_copy(ev_i_h.at[wid], idx_ev)
    pltpu.sync_copy(cls_i_h.at[wid], idx_cls)
    pltpu.sync_copy(time_i_h.at[wid], idx_time)
    pltpu.sync_copy(pos_i_h.at[wid], idx_pos)
    pltpu.sync_copy(neg_i_h.at[wid], idx_neg)

    def srcs():
        return (ev0, ev1, ev2, clsemb, temb, ctxw0, ctxw1, ctxw0, ctxw1)

    def idx_for(slot, step):
        if slot < 3:
            return idx_ev.at[step]
        if slot == 3:
            return idx_cls.at[step]
        if slot == 4:
            return idx_time.at[step]
        if slot < 7:
            return idx_pos.at[step]
        return idx_neg.at[step]

    def fire(step, bufs):
        sem = bufs[-1]
        for slot, src in enumerate(srcs()):
            pltpu.async_copy(src.at[idx_for(slot, step)], bufs[slot], sem)

    def drain(bufs):
        sem = bufs[-1]
        for slot, src in enumerate(srcs()):
            pltpu.make_async_copy(src.at[idx_for(slot, 0)], bufs[slot], sem).wait()

    def compute(step, gphase4, bufs, sbuf):
        (ev_g0, ev_g1, ev_g2, cls_r, time_r,
         pos_w0, pos_w1, neg_w0, neg_w1, _) = bufs
        for r in range(_C):
            u_regs = _load_u_regs((ev_g0, ev_g1, ev_g2), cls_r, time_r, r)
            d0 = (gphase4 * _C + r) * 21
            acc = _dot400w(u_regs, (pos_w0, pos_w1), r)
            _store_acc(sbuf, jnp.int32(d0), acc)

            @pl.loop(0, _K)
            def _neg(j):
                accn = _dot400w(u_regs, (neg_w0, neg_w1), r * _K + j)
                _store_acc(sbuf, d0 + 1 + j, accn)

    fire(0, sets[0])
    fire(1, sets[1])

    @pl.loop(0, _STEPS, step=2 * _QS)
    def _grp8(s8):
        for gphase in range(2):
            s4 = s8 + gphase * _QS
            sbuf, ssem = (sbufA, ssemA) if gphase == 0 else (sbufB, ssemB)

            @pl.when(s4 >= 2 * _QS)
            def _wait_prev():
                pltpu.make_async_copy(sbuf, out.at[pl.ds(0, _GRP_F32)], ssem).wait()

            for phase4 in range(_QS):
                step = s4 + phase4
                bufs = sets[phase4 % 2]
                drain(bufs)
                compute(step, phase4, bufs, sbuf)

                @pl.when(step + 2 < _STEPS)
                def _refire():
                    fire(step + 2, bufs)

            pltpu.async_copy(
                sbuf, out.at[pl.ds(wid * _W_SPAN + (s4 // _QS) * _GRP_F32,
                                   _GRP_F32)], ssem)

    pltpu.make_async_copy(sbufA, out.at[pl.ds(0, _GRP_F32)], ssemA).wait()
    pltpu.make_async_copy(sbufB, out.at[pl.ds(0, _GRP_F32)], ssemB).wait()


def _buf_set():
    return [
        pltpu.VMEM((_C, 128), jnp.float32),
        pltpu.VMEM((_C, 128), jnp.float32),
        pltpu.VMEM((_C, 128), jnp.float32),
        pltpu.VMEM((_C, _D_CLS), jnp.float32),
        pltpu.VMEM((_C, _D_TIME), jnp.float32),
        pltpu.VMEM((_C, 128), jnp.float32),
        pltpu.VMEM((_C, 128), jnp.float32),
        pltpu.VMEM((_C * _K, 128), jnp.float32),
        pltpu.VMEM((_C * _K, 128), jnp.float32),
    ]


_sc_scores = pl.kernel(
    _sc_body,
    out_type=jax.ShapeDtypeStruct((_OUT_N,), jnp.float32),
    mesh=plsc.VectorSubcoreMesh(core_axis_name="c", subcore_axis_name="s"),
    compiler_params=pltpu.CompilerParams(use_tc_tiling_on_sc=False,
                                         needs_layout_passes=False),
    scratch_types=[
        pltpu.VMEM((_STEPS, _C), jnp.int32),
        pltpu.VMEM((_STEPS, _C), jnp.int32),
        pltpu.VMEM((_STEPS, _C), jnp.int32),
        pltpu.VMEM((_STEPS, _C), jnp.int32),
        pltpu.VMEM((_STEPS, _C * _K), jnp.int32),
    ] + _buf_set() + _buf_set() + [
        pltpu.VMEM((_GRP_F32,), jnp.float32),
        pltpu.VMEM((_GRP_F32,), jnp.float32),
        pltpu.SemaphoreType.DMA,
        pltpu.SemaphoreType.DMA,
        pltpu.SemaphoreType.DMA,
        pltpu.SemaphoreType.DMA,
    ],
)


_TR_BLK = 1024


def _pack_words(xb, ncols):
    xb = jnp.concatenate(
        [xb, jnp.zeros((_TR_BLK, 512 - ncols), jnp.bfloat16)], axis=1)
    lo = lax.convert_element_type(
        lax.bitcast_convert_type(xb[:, :256], jnp.uint16), jnp.uint32)
    hi = lax.convert_element_type(
        lax.bitcast_convert_type(xb[:, 256:], jnp.uint16), jnp.uint32)
    return lax.bitcast_convert_type(lo | (hi << 16), jnp.float32)


def _tr_split_body(ev_ref, ctx_ref, e0_ref, e1_ref, e2_ref, c0_ref, c1_ref):
    et = ev_ref[...].T
    e0_ref[...] = et[:, 0:128]
    e1_ref[...] = et[:, 128:256]
    e2_ref[...] = jnp.concatenate(
        [et[:, 256:_D_EV], jnp.zeros((_TR_BLK, 128 - (_D_EV - 256)), jnp.float32)],
        axis=1)
    cw = _pack_words(ctx_ref[...].T.astype(jnp.bfloat16), _D_U)
    c0_ref[...] = cw[:, :128]
    c1_ref[...] = cw[:, 128:]


def _relayout_tables(evt_view, ctxt_view):
    grid = (_N_CTX + _TR_BLK - 1) // _TR_BLK
    return pl.pallas_call(
        _tr_split_body,
        grid=(grid,),
        in_specs=[
            pl.BlockSpec((_D_EV, _TR_BLK), lambda j: (0, j)),
            pl.BlockSpec((_D_U, _TR_BLK), lambda j: (0, j)),
        ],
        out_specs=[pl.BlockSpec((_TR_BLK, 128), lambda j: (j, 0))] * 5,
        out_shape=[jax.ShapeDtypeStruct((_N_EV, 128), jnp.float32)] * 3
        + [jax.ShapeDtypeStruct((_N_CTX, 128), jnp.float32)] * 2,
    )(evt_view, ctxt_view)


def _log_sigmoid(x):
    return jnp.minimum(x, 0.0) - jnp.log1p(jnp.exp(-jnp.abs(x)))


_RED_GRID = 8
_RED_ROWS = _OUT_N // 128 // _RED_GRID


def _reduce_body(s_ref, o_ref):
    blk = pl.program_id(0)
    x = s_ref[...]
    kcol = lax.broadcasted_iota(jnp.int32, (128, 8), 1)
    krow = lax.broadcasted_iota(jnp.int32, (128, 8), 0)
    m = (krow // 16 == kcol).astype(jnp.float32)
    s = jnp.dot(x, m, preferred_element_type=jnp.float32)
    i2 = lax.broadcasted_iota(jnp.int32, (_RED_ROWS, 8), 0)
    k2 = lax.broadcasted_iota(jnp.int32, (_RED_ROWS, 8), 1)
    d = (blk * _RED_ROWS + i2) * 8 + k2
    is_pos = (d % 21) == 0
    part = jnp.sum(jnp.where(is_pos, _log_sigmoid(s), _log_sigmoid(-s)))

    @pl.when(blk == 0)
    def _init():
        o_ref[0, 0] = 0.0

    o_ref[0, 0] += part

    @pl.when(blk == _RED_GRID - 1)
    def _fin():
        o_ref[0, 0] = -o_ref[0, 0] / jnp.float32(_B)


_reduce_loss = pl.pallas_call(
    _reduce_body,
    grid=(_RED_GRID,),
    in_specs=[pl.BlockSpec((_RED_ROWS, 128), lambda j: (j, 0))],
    out_specs=pl.BlockSpec(memory_space=pltpu.SMEM),
    out_shape=jax.ShapeDtypeStruct((1, 1), jnp.float32),
)


def kernel(ev_idx, cls_idx, time_idx, pos_idx, neg_idx,
           event_emb, class_emb, time_emb, context_emb):
    ev3 = ev_idx.astype(jnp.int32).reshape(_NW, _STEPS, _C)
    cls3 = cls_idx.astype(jnp.int32).reshape(_NW, _STEPS, _C)
    time3 = time_idx.astype(jnp.int32).reshape(_NW, _STEPS, _C)
    pos3 = pos_idx.astype(jnp.int32).reshape(_NW, _STEPS, _C)
    neg3 = neg_idx.astype(jnp.int32).reshape(_NW, _STEPS, _C * _K)
    ev0, ev1, ev2, cw0, cw1 = _relayout_tables(event_emb.T, context_emb.T)
    parts = _sc_scores(ev3, cls3, time3, pos3, neg3,
                       ev0, ev1, ev2, class_emb, time_emb, cw0, cw1)
    return _reduce_loss(parts.reshape(_OUT_N // 128, 128))[0, 0]


# speedup vs baseline: 1.0482x; 1.0482x over previous
import jax
import jax.numpy as jnp
import numpy as _np
from jax import lax
from jax.experimental import pallas as pl
from jax.experimental.pallas import tpu as pltpu
from jax.experimental.pallas import tpu_sc as plsc

_B = 16384
_K = 20
_D_EV = 300
_D_CLS = 64
_D_TIME = 36
_D_U = 400
_NCHUNK = _D_U // 16
_N_EV = 100000
_N_CTX = 100000
_NWORD_CHUNKS = 16
_NB_CHUNKS = 9

_NC = 2
_NS = 16
_NW = _NC * _NS
_BW = _B // _NW
_C = 4
_STEPS = _BW // _C


def _lanes():
    return lax.iota(jnp.int32, 16)


def _load_u_regs(ev_g, cls_r, time_r, r):
    li = _lanes()
    row = jnp.full((16,), r, jnp.int32)
    regs = []
    for c in range(_NCHUNK):
        d0 = c * 16
        if c == 18:
            a = plsc.load_gather(ev_g[2], [row, d0 - 256 + li])
            b = plsc.load_gather(cls_r, [row, jnp.maximum(d0 + li - _D_EV, 0)])
            regs.append(jnp.where(li < 12, a, b))
        elif c == 22:
            a = plsc.load_gather(cls_r, [row, jnp.minimum(d0 - _D_EV + li, _D_CLS - 1)])
            b = plsc.load_gather(time_r, [row, jnp.maximum(d0 + li - (_D_EV + _D_CLS), 0)])
            regs.append(jnp.where(li < 12, a, b))
        elif d0 + 16 <= _D_EV:
            regs.append(plsc.load_gather(ev_g[c // 8], [row, (c % 8) * 16 + li]))
        elif d0 + 16 <= _D_EV + _D_CLS:
            regs.append(plsc.load_gather(cls_r, [row, d0 - _D_EV + li]))
        else:
            regs.append(plsc.load_gather(time_r, [row, d0 - (_D_EV + _D_CLS) + li]))
    return regs


def _dot400w(u_regs, w_g, vrow):
    li = _lanes()
    row = jnp.full((16,), vrow, jnp.int32)
    accs = [jnp.zeros((16,), jnp.float32) for _ in range(4)]
    for w in range(_NWORD_CHUNKS):
        wv = plsc.load_gather(w_g[w // 8], [row, (w % 8) * 16 + li])
        wi = plsc.bitcast(wv, jnp.int32)
        a = plsc.bitcast(wi << 16, jnp.float32)
        accs[w % 4] = accs[w % 4] + u_regs[w] * a
        if w < _NB_CHUNKS:
            accs[(w + 2) % 4] = accs[(w + 2) % 4] + u_regs[16 + w] * wv
    return (accs[0] + accs[1]) + (accs[2] + accs[3])


_QS = 4
_GRP_F32 = _QS * _C * 21 * 16
_W_SPAN = _BW * 21 * 16
_OUT_N = _B * 21 * 16


def _store_acc(sbuf, d, acc):
    li = _lanes()
    plsc.store_scatter(sbuf, [d * 16 + li], acc)


def _sc_body(ev_i_h, cls_i_h, time_i_h, pos_i_h, neg_i_h,
             ev0, ev1, ev2, clsemb, temb, ctxw0, ctxw1, out,
             idx_ev, idx_cls, idx_time, idx_pos, idx_neg,
             *bufs_flat):
    sbufA = bufs_flat[-6]
    sbufB = bufs_flat[-5]
    ssemA = bufs_flat[-4]
    ssemB = bufs_flat[-3]
    sem0 = bufs_flat[-2]
    sem1 = bufs_flat[-1]
    nper = (len(bufs_flat) - 6) // 2
    sets = (tuple(bufs_flat[:nper]) + (sem0,),
            tuple(bufs_flat[nper:2 * nper]) + (sem1,))

    cid = lax.axis_index("c")
    sid = lax.axis_index("s")
    wid = sid * _NC + cid

    pltpu.sync_copy(ev_i_h.at[wid], idx_ev)
    pltpu.sync_copy(cls_i_h.at[wid], idx_cls)
    pltpu.sync_copy(time_i_h.at[wid], idx_time)
    pltpu.sync_copy(pos_i_h.at[wid], idx_pos)
    pltpu.sync_copy(neg_i_h.at[wid], idx_neg)

    def srcs():
        return (ev0, ev1, ev2, clsemb, temb, ctxw0, ctxw1, ctxw0, ctxw1)

    def idx_for(slot, step):
        if slot < 3:
            return idx_ev.at[step]
        if slot == 3:
            return idx_cls.at[step]
        if slot == 4:
            return idx_time.at[step]
        if slot < 7:
            return idx_pos.at[step]
        return idx_neg.at[step]

    def fire(step, bufs):
        sem = bufs[-1]
        for slot, src in enumerate(srcs()):
            pltpu.async_copy(src.at[idx_for(slot, step)], bufs[slot], sem)

    def drain(bufs):
        sem = bufs[-1]
        for slot, src in enumerate(srcs()):
            pltpu.make_async_copy(src.at[idx_for(slot, 0)], bufs[slot], sem).wait()

    def compute(step, gphase4, bufs, sbuf):
        (ev_g0, ev_g1, ev_g2, cls_r, time_r,
         pos_w0, pos_w1, neg_w0, neg_w1, _) = bufs
        for r in range(_C):
            u_regs = _load_u_regs((ev_g0, ev_g1, ev_g2), cls_r, time_r, r)
            d0 = (gphase4 * _C + r) * 21
            acc = _dot400w(u_regs, (pos_w0, pos_w1), r)
            _store_acc(sbuf, jnp.int32(d0), acc)

            @pl.loop(0, _K)
            def _neg(j):
                accn = _dot400w(u_regs, (neg_w0, neg_w1), r * _K + j)
                _store_acc(sbuf, d0 + 1 + j, accn)

    fire(0, sets[0])
    fire(1, sets[1])

    @pl.loop(0, _STEPS, step=2 * _QS)
    def _grp8(s8):
        for gphase in range(2):
            s4 = s8 + gphase * _QS
            sbuf, ssem = (sbufA, ssemA) if gphase == 0 else (sbufB, ssemB)

            @pl.when(s4 >= 2 * _QS)
            def _wait_prev():
                pltpu.make_async_copy(sbuf, out.at[pl.ds(0, _GRP_F32)], ssem).wait()

            for phase4 in range(_QS):
                step = s4 + phase4
                bufs = sets[phase4 % 2]
                drain(bufs)
                compute(step, phase4, bufs, sbuf)

                @pl.when(step + 2 < _STEPS)
                def _refire():
                    fire(step + 2, bufs)

            pltpu.async_copy(
                sbuf, out.at[pl.ds(wid * _W_SPAN + (s4 // _QS) * _GRP_F32,
                                   _GRP_F32)], ssem)

    pltpu.make_async_copy(sbufA, out.at[pl.ds(0, _GRP_F32)], ssemA).wait()
    pltpu.make_async_copy(sbufB, out.at[pl.ds(0, _GRP_F32)], ssemB).wait()


def _buf_set():
    return [
        pltpu.VMEM((_C, 128), jnp.float32),
        pltpu.VMEM((_C, 128), jnp.float32),
        pltpu.VMEM((_C, 128), jnp.float32),
        pltpu.VMEM((_C, _D_CLS), jnp.float32),
        pltpu.VMEM((_C, _D_TIME), jnp.float32),
        pltpu.VMEM((_C, 128), jnp.float32),
        pltpu.VMEM((_C, 128), jnp.float32),
        pltpu.VMEM((_C * _K, 128), jnp.float32),
        pltpu.VMEM((_C * _K, 128), jnp.float32),
    ]


_sc_scores = pl.kernel(
    _sc_body,
    out_type=jax.ShapeDtypeStruct((_OUT_N,), jnp.float32),
    mesh=plsc.VectorSubcoreMesh(core_axis_name="c", subcore_axis_name="s"),
    compiler_params=pltpu.CompilerParams(use_tc_tiling_on_sc=False,
                                         needs_layout_passes=False),
    scratch_types=[
        pltpu.VMEM((_STEPS, _C), jnp.int32),
        pltpu.VMEM((_STEPS, _C), jnp.int32),
        pltpu.VMEM((_STEPS, _C), jnp.int32),
        pltpu.VMEM((_STEPS, _C), jnp.int32),
        pltpu.VMEM((_STEPS, _C * _K), jnp.int32),
    ] + _buf_set() + _buf_set() + [
        pltpu.VMEM((_GRP_F32,), jnp.float32),
        pltpu.VMEM((_GRP_F32,), jnp.float32),
        pltpu.SemaphoreType.DMA,
        pltpu.SemaphoreType.DMA,
        pltpu.SemaphoreType.DMA,
        pltpu.SemaphoreType.DMA,
    ],
)


_TR_BLK = 1024


def _pack_words(xb, ncols):
    xb = jnp.concatenate(
        [xb, jnp.zeros((_TR_BLK, 512 - ncols), jnp.bfloat16)], axis=1)
    lo = lax.convert_element_type(
        lax.bitcast_convert_type(xb[:, :256], jnp.uint16), jnp.uint32)
    hi = lax.convert_element_type(
        lax.bitcast_convert_type(xb[:, 256:], jnp.uint16), jnp.uint32)
    return lax.bitcast_convert_type(lo | (hi << 16), jnp.float32)


def _tr_split_body(ev_ref, ctx_ref, e0_ref, e1_ref, e2_ref, c0_ref, c1_ref):
    et = ev_ref[...].T
    e0_ref[...] = et[:, 0:128]
    e1_ref[...] = et[:, 128:256]
    e2_ref[...] = jnp.concatenate(
        [et[:, 256:_D_EV], jnp.zeros((_TR_BLK, 128 - (_D_EV - 256)), jnp.float32)],
        axis=1)
    cw = _pack_words(ctx_ref[...].T.astype(jnp.bfloat16), _D_U)
    c0_ref[...] = cw[:, :128]
    c1_ref[...] = cw[:, 128:]


def _relayout_tables(evt_view, ctxt_view):
    grid = (_N_CTX + _TR_BLK - 1) // _TR_BLK
    return pl.pallas_call(
        _tr_split_body,
        grid=(grid,),
        in_specs=[
            pl.BlockSpec((_D_EV, _TR_BLK), lambda j: (0, j)),
            pl.BlockSpec((_D_U, _TR_BLK), lambda j: (0, j)),
        ],
        out_specs=[pl.BlockSpec((_TR_BLK, 128), lambda j: (j, 0))] * 5,
        out_shape=[jax.ShapeDtypeStruct((_N_EV, 128), jnp.float32)] * 3
        + [jax.ShapeDtypeStruct((_N_CTX, 128), jnp.float32)] * 2,
    )(evt_view, ctxt_view)


def _log_sigmoid(x):
    return jnp.minimum(x, 0.0) - jnp.log1p(jnp.exp(-jnp.abs(x)))


_RED_GRID = 8
_RED_ROWS = _OUT_N // 128 // _RED_GRID


_SIGN_NP = _np.where(
    _np.arange(_RED_ROWS * 8).reshape(_RED_ROWS, 8) % 21 == 0, 1.0, -1.0
).astype(_np.float32)


def _reduce_body(s_ref, sign_ref, o_ref):
    blk = pl.program_id(0)
    x = s_ref[...]
    kcol = lax.broadcasted_iota(jnp.int32, (128, 8), 1)
    krow = lax.broadcasted_iota(jnp.int32, (128, 8), 0)
    m = (krow // 16 == kcol).astype(jnp.float32)
    s = jnp.dot(x, m, preferred_element_type=jnp.float32)
    part = jnp.sum(_log_sigmoid(s * sign_ref[...]))

    @pl.when(blk == 0)
    def _init():
        o_ref[0, 0] = 0.0

    o_ref[0, 0] += part

    @pl.when(blk == _RED_GRID - 1)
    def _fin():
        o_ref[0, 0] = -o_ref[0, 0] / jnp.float32(_B)


_reduce_loss = pl.pallas_call(
    _reduce_body,
    grid=(_RED_GRID,),
    in_specs=[pl.BlockSpec((_RED_ROWS, 128), lambda j: (j, 0)),
              pl.BlockSpec((_RED_ROWS, 8), lambda j: (0, 0))],
    out_specs=pl.BlockSpec(memory_space=pltpu.SMEM),
    out_shape=jax.ShapeDtypeStruct((1, 1), jnp.float32),
)


def kernel(ev_idx, cls_idx, time_idx, pos_idx, neg_idx,
           event_emb, class_emb, time_emb, context_emb):
    ev3 = ev_idx.astype(jnp.int32).reshape(_NW, _STEPS, _C)
    cls3 = cls_idx.astype(jnp.int32).reshape(_NW, _STEPS, _C)
    time3 = time_idx.astype(jnp.int32).reshape(_NW, _STEPS, _C)
    pos3 = pos_idx.astype(jnp.int32).reshape(_NW, _STEPS, _C)
    neg3 = neg_idx.astype(jnp.int32).reshape(_NW, _STEPS, _C * _K)
    ev0, ev1, ev2, cw0, cw1 = _relayout_tables(event_emb.T, context_emb.T)
    parts = _sc_scores(ev3, cls3, time3, pos3, neg3,
                       ev0, ev1, ev2, class_emb, time_emb, cw0, cw1)
    return _reduce_loss(parts.reshape(_OUT_N // 128, 128),
                        jnp.asarray(_SIGN_NP))[0, 0]
